# Initial kernel scaffold; baseline (speedup 1.0000x reference)
import jax
import jax.numpy as jnp
from jax.experimental import pallas as pl


def kernel(word, pos1, pos2, word_table, pos1_table, pos2_table):
    raise NotImplementedError("write your pallas kernel here")


# speedup vs baseline: 6.9687x; 6.9687x over previous
import functools

import jax
import jax.numpy as jnp
from jax import lax
from jax.experimental import pallas as pl
from jax.experimental.pallas import tpu as pltpu
from jax.experimental.pallas import tpu_sc as plsc

B = 4096
L = 200
WDIM = 64
PDIM = 16
ODIM = WDIM + 2 * PDIM
N = B * L
NC = 2
NS = 16
NW = NC * NS
CHUNK = N // NW
G = 4
GT = 128
T = G * GT
STEPS = CHUNK // T


def _emb_body(word_hbm, pos1_hbm, pos2_hbm, wtab_hbm, p1tab_hbm, p2tab_hbm,
              out_hbm, widx_v, p1idx_v, p2idx_v, wrows_v, p1rows_v, p2rows_v,
              sem):
    c = lax.axis_index("c")
    s = lax.axis_index("s")
    wid = s * NC + c
    row0 = wid * (CHUNK // GT)

    def step(i, carry):
        base = wid * CHUNK + i * T
        r = row0 + i * G
        pltpu.sync_copy(word_hbm.at[pl.ds(r, G)], widx_v)
        pltpu.sync_copy(pos1_hbm.at[pl.ds(r, G)], p1idx_v)
        pltpu.sync_copy(pos2_hbm.at[pl.ds(r, G)], p2idx_v)
        copies = []
        for j in range(G):
            copies.append(pltpu.async_copy(
                wtab_hbm.at[widx_v.at[j]],
                wrows_v.at[pl.ds(j * GT, GT)], sem))
            copies.append(pltpu.async_copy(
                p1tab_hbm.at[p1idx_v.at[j]],
                p1rows_v.at[pl.ds(j * GT, GT)], sem))
            copies.append(pltpu.async_copy(
                p2tab_hbm.at[p2idx_v.at[j]],
                p2rows_v.at[pl.ds(j * GT, GT)], sem))
        for cp in copies:
            cp.wait()
        pltpu.sync_copy(wrows_v, out_hbm.at[pl.ds(base, T), pl.ds(0, WDIM)])
        pltpu.sync_copy(p1rows_v, out_hbm.at[pl.ds(base, T), pl.ds(WDIM, PDIM)])
        pltpu.sync_copy(p2rows_v,
                        out_hbm.at[pl.ds(base, T), pl.ds(WDIM + PDIM, PDIM)])
        return carry

    lax.fori_loop(0, STEPS, step, 0)


@functools.partial(jax.jit, static_argnames=())
def _run(word_flat, pos1_flat, pos2_flat, word_table, p1_tab, p2_tab):
    mesh = plsc.VectorSubcoreMesh(core_axis_name="c", subcore_axis_name="s")
    f = pl.kernel(
        _emb_body,
        mesh=mesh,
        compiler_params=pltpu.CompilerParams(use_tc_tiling_on_sc=False),
        out_type=jax.ShapeDtypeStruct((N, ODIM), jnp.float32),
        scratch_types=[
            pltpu.VMEM((G, GT), jnp.int32),
            pltpu.VMEM((G, GT), jnp.int32),
            pltpu.VMEM((G, GT), jnp.int32),
            pltpu.VMEM((T, WDIM), jnp.float32),
            pltpu.VMEM((T, PDIM), jnp.float32),
            pltpu.VMEM((T, PDIM), jnp.float32),
            pltpu.SemaphoreType.DMA,
        ],
    )
    return f(word_flat, pos1_flat, pos2_flat, word_table, p1_tab, p2_tab)


def kernel(word, pos1, pos2, word_table, pos1_table, pos2_table):
    word_flat = word.reshape(N // GT, GT).astype(jnp.int32)
    pos1_flat = pos1.reshape(N // GT, GT).astype(jnp.int32)
    pos2_flat = pos2.reshape(N // GT, GT).astype(jnp.int32)
    p1_tab = pos1_table.at[0].set(0.0)
    p2_tab = pos2_table.at[0].set(0.0)
    out = _run(word_flat, pos1_flat, pos2_flat, word_table, p1_tab, p2_tab)
    return out.reshape(B, L, ODIM)


# speedup vs baseline: 7.3778x; 1.0587x over previous
import functools

import jax
import jax.numpy as jnp
from jax import lax
from jax.experimental import pallas as pl
from jax.experimental.pallas import tpu as pltpu
from jax.experimental.pallas import tpu_sc as plsc

B = 4096
L = 200
WDIM = 64
PDIM = 16
ODIM = WDIM + 2 * PDIM
N = B * L
NC = 2
NS = 16
NW = NC * NS
CHUNK = N // NW
T = 512
STEPS = CHUNK // T


def _emb_body(word_hbm, pos1_hbm, pos2_hbm, wtab_hbm, p1tab_hbm, p2tab_hbm,
              out_hbm, widx_v, p1idx_v, p2idx_v, wrows_v, p1rows_v, p2rows_v,
              sem_i, sem_g, sem_o):
    c = lax.axis_index("c")
    s = lax.axis_index("s")
    wid = s * NC + c
    base0 = wid * CHUNK

    def idx_copies(i, p):
        base = base0 + i * T
        return [
            pltpu.make_async_copy(word_hbm.at[pl.ds(base, T)],
                                  widx_v.at[p], sem_i.at[p]),
            pltpu.make_async_copy(pos1_hbm.at[pl.ds(base, T)],
                                  p1idx_v.at[p], sem_i.at[p]),
            pltpu.make_async_copy(pos2_hbm.at[pl.ds(base, T)],
                                  p2idx_v.at[p], sem_i.at[p]),
        ]

    def gather_copies(p):
        return [
            pltpu.make_async_copy(wtab_hbm.at[widx_v.at[p]],
                                  wrows_v.at[p], sem_g.at[p]),
            pltpu.make_async_copy(p1tab_hbm.at[p1idx_v.at[p]],
                                  p1rows_v.at[p], sem_g.at[p]),
            pltpu.make_async_copy(p2tab_hbm.at[p2idx_v.at[p]],
                                  p2rows_v.at[p], sem_g.at[p]),
        ]

    def out_copies(i, p):
        base = base0 + i * T
        return [
            pltpu.make_async_copy(
                wrows_v.at[p],
                out_hbm.at[pl.ds(base, T), pl.ds(0, WDIM)], sem_o.at[p]),
            pltpu.make_async_copy(
                p1rows_v.at[p],
                out_hbm.at[pl.ds(base, T), pl.ds(WDIM, PDIM)], sem_o.at[p]),
            pltpu.make_async_copy(
                p2rows_v.at[p],
                out_hbm.at[pl.ds(base, T), pl.ds(WDIM + PDIM, PDIM)],
                sem_o.at[p]),
        ]

    def fire(copies):
        for cp in copies:
            cp.start()

    def drain(copies):
        for cp in copies:
            cp.wait()

    fire(idx_copies(0, 0))
    fire(idx_copies(1, 1))
    drain(idx_copies(0, 0))
    fire(gather_copies(0))

    def step(i, carry):
        p = i & 1
        q = 1 - p
        drain(gather_copies(q))
        fire(out_copies(i - 1, q))

        @pl.when(i + 1 < STEPS)
        def _():
            fire(idx_copies(i + 1, q))

        drain(idx_copies(i, p))

        @pl.when(i >= 2)
        def _():
            drain(out_copies(i - 2, p))

        fire(gather_copies(p))
        return carry

    lax.fori_loop(1, STEPS, step, 0)

    qe = (STEPS - 1) & 1
    drain(gather_copies(qe))
    fire(out_copies(STEPS - 1, qe))
    drain(out_copies(STEPS - 2, 1 - qe))
    drain(out_copies(STEPS - 1, qe))


@jax.jit
def _run(word_flat, pos1_flat, pos2_flat, word_table, p1_tab, p2_tab):
    mesh = plsc.VectorSubcoreMesh(core_axis_name="c", subcore_axis_name="s")
    f = pl.kernel(
        _emb_body,
        mesh=mesh,
        compiler_params=pltpu.CompilerParams(use_tc_tiling_on_sc=False),
        out_type=jax.ShapeDtypeStruct((N, ODIM), jnp.float32),
        scratch_types=[
            pltpu.VMEM((2, T), jnp.int32),
            pltpu.VMEM((2, T), jnp.int32),
            pltpu.VMEM((2, T), jnp.int32),
            pltpu.VMEM((2, T, WDIM), jnp.float32),
            pltpu.VMEM((2, T, PDIM), jnp.float32),
            pltpu.VMEM((2, T, PDIM), jnp.float32),
            pltpu.SemaphoreType.DMA((2,)),
            pltpu.SemaphoreType.DMA((2,)),
            pltpu.SemaphoreType.DMA((2,)),
        ],
    )
    return f(word_flat, pos1_flat, pos2_flat, word_table, p1_tab, p2_tab)


def kernel(word, pos1, pos2, word_table, pos1_table, pos2_table):
    word_flat = word.reshape(N).astype(jnp.int32)
    pos1_flat = pos1.reshape(N).astype(jnp.int32)
    pos2_flat = pos2.reshape(N).astype(jnp.int32)
    p1_tab = pos1_table.at[0].set(0.0)
    p2_tab = pos2_table.at[0].set(0.0)
    out = _run(word_flat, pos1_flat, pos2_flat, word_table, p1_tab, p2_tab)
    return out.reshape(B, L, ODIM)
